# baseline (speedup 1.0000x reference)
import functools

import jax
import jax.numpy as jnp
from jax import lax
from jax.experimental import pallas as pl
from jax.experimental.pallas import tpu as pltpu
from jax.experimental.pallas import tpu_sc as plsc

L = 1024
E = 65536
NC, NS = 2, 16
EPC = E // NS
CHUNK = 128
NCHUNK = EPC // CHUNK
SLICE = (L * L) // NS
ZB = 8192


def _build_w_body(d1_hbm, s1_hbm, w1_hbm, d2_hbm, s2_hbm, w2_hbm, out_hbm,
                  d1_v, s1_v, v1_v, d2_v, s2_v, v2_v, val_v, idx_v, zbuf_v, w_sh):
    c = lax.axis_index("c")
    s = lax.axis_index("s")
    base = s * EPC

    pltpu.sync_copy(d1_hbm.at[pl.ds(base, EPC)], d1_v)
    pltpu.sync_copy(s1_hbm.at[pl.ds(base, EPC)], s1_v)
    pltpu.sync_copy(w1_hbm.at[pl.ds(base, EPC)], v1_v)
    pltpu.sync_copy(d2_hbm.at[pl.ds(base, EPC)], d2_v)
    pltpu.sync_copy(s2_hbm.at[pl.ds(base, EPC)], s2_v)
    pltpu.sync_copy(w2_hbm.at[pl.ds(base, EPC)], v2_v)

    for k in range(ZB // 16):
        zbuf_v[pl.ds(k * 16, 16)] = jnp.zeros((16,), jnp.float32)
    for k in range(SLICE // ZB):
        pltpu.sync_copy(zbuf_v, w_sh.at[pl.ds(s * SLICE + k * ZB, ZB)])

    on1 = c == 0
    for j in range(NCHUNK):
        for i in range(CHUNK // 16):
            t = j * CHUNK + i * 16
            srcv = jnp.where(on1, s1_v[pl.ds(t, 16)], s2_v[pl.ds(t, 16)])
            dstv = jnp.where(on1, d1_v[pl.ds(t, 16)], d2_v[pl.ds(t, 16)])
            idx_v[j, pl.ds(i * 16, 16)] = srcv * L + dstv
            val_v[pl.ds(t, 16)] = jnp.where(
                on1, v1_v[pl.ds(t, 16)], v2_v[pl.ds(t, 16)])
    plsc.subcore_barrier()

    for j in range(NCHUNK):
        pltpu.sync_copy(val_v.at[pl.ds(j * CHUNK, CHUNK)],
                        w_sh.at[idx_v.at[j]], add=True)
    plsc.subcore_barrier()

    pltpu.sync_copy(w_sh.at[pl.ds(s * SLICE, SLICE)],
                    out_hbm.at[c, pl.ds(s * SLICE, SLICE)])


@functools.partial(
    pl.kernel,
    out_type=jax.ShapeDtypeStruct((NC, L * L), jnp.float32),
    mesh=plsc.VectorSubcoreMesh(core_axis_name="c", subcore_axis_name="s"),
    scratch_types=[
        pltpu.VMEM((EPC,), jnp.int32),
        pltpu.VMEM((EPC,), jnp.int32),
        pltpu.VMEM((EPC,), jnp.float32),
        pltpu.VMEM((EPC,), jnp.int32),
        pltpu.VMEM((EPC,), jnp.int32),
        pltpu.VMEM((EPC,), jnp.float32),
        pltpu.VMEM((EPC,), jnp.float32),
        pltpu.VMEM((NCHUNK, CHUNK), jnp.int32),
        pltpu.VMEM((ZB,), jnp.float32),
        pltpu.VMEM_SHARED((L * L,), jnp.float32),
    ],
)
def _build_w(d1_hbm, s1_hbm, w1_hbm, d2_hbm, s2_hbm, w2_hbm, out_hbm,
             d1_v, s1_v, v1_v, d2_v, s2_v, v2_v, val_v, idx_v, zbuf_v, w_sh):
    _build_w_body(d1_hbm, s1_hbm, w1_hbm, d2_hbm, s2_hbm, w2_hbm, out_hbm,
                  d1_v, s1_v, v1_v, d2_v, s2_v, v2_v, val_v, idx_v, zbuf_v, w_sh)


def _mlp_body(x_ref, w_ref, b1_ref, b2_ref, o_ref):
    h = jnp.dot(x_ref[...], w_ref[0], preferred_element_type=jnp.float32)
    h = jnp.maximum(h + b1_ref[...], 0.0)
    o_ref[...] = jnp.dot(h, w_ref[1], preferred_element_type=jnp.float32) + b2_ref[...]


def kernel(input_tensor, edge_index1, weights1, bias1,
           edge_index2, weights2, bias2):
    ei1 = edge_index1.astype(jnp.int32)
    ei2 = edge_index2.astype(jnp.int32)
    w12 = _build_w(ei1[0], ei1[1], weights1,
                   ei2[0], ei2[1], weights2).reshape(NC, L, L)

    out = pl.pallas_call(
        _mlp_body,
        out_shape=jax.ShapeDtypeStruct((input_tensor.shape[0], L), jnp.float32),
    )(input_tensor, w12, bias1.reshape(1, L), bias2.reshape(1, L))
    return out


# speedup vs baseline: 1.0387x; 1.0387x over previous
import functools

import jax
import jax.numpy as jnp
from jax import lax
from jax.experimental import pallas as pl
from jax.experimental.pallas import tpu as pltpu
from jax.experimental.pallas import tpu_sc as plsc

L = 1024
E = 65536
NC, NS = 2, 16
EPC = E // NS
CHUNK = 128
NCHUNK = EPC // CHUNK
SLICE = (L * L) // NS
ZB = 8192


def _build_w_body(d1_hbm, s1_hbm, w1_hbm, d2_hbm, s2_hbm, w2_hbm, out_hbm,
                  d1_v, s1_v, v1_v, d2_v, s2_v, v2_v, val_v, idx_v, zbuf_v, w_sh):
    c = lax.axis_index("c")
    s = lax.axis_index("s")
    base = s * EPC

    pltpu.sync_copy(d1_hbm.at[pl.ds(base, EPC)], d1_v)
    pltpu.sync_copy(s1_hbm.at[pl.ds(base, EPC)], s1_v)
    pltpu.sync_copy(w1_hbm.at[pl.ds(base, EPC)], v1_v)
    pltpu.sync_copy(d2_hbm.at[pl.ds(base, EPC)], d2_v)
    pltpu.sync_copy(s2_hbm.at[pl.ds(base, EPC)], s2_v)
    pltpu.sync_copy(w2_hbm.at[pl.ds(base, EPC)], v2_v)

    for k in range(ZB // 16):
        zbuf_v[pl.ds(k * 16, 16)] = jnp.zeros((16,), jnp.float32)
    for k in range(SLICE // ZB):
        pltpu.sync_copy(zbuf_v, w_sh.at[pl.ds(s * SLICE + k * ZB, ZB)])

    on1 = c == 0
    for t in range(0, EPC, 16):
        srcv = jnp.where(on1, s1_v[pl.ds(t, 16)], s2_v[pl.ds(t, 16)])
        dstv = jnp.where(on1, d1_v[pl.ds(t, 16)], d2_v[pl.ds(t, 16)])
        idx_v[pl.ds(t, 16)] = srcv * L + dstv
        val_v[pl.ds(t, 16)] = jnp.where(
            on1, v1_v[pl.ds(t, 16)], v2_v[pl.ds(t, 16)])
    plsc.subcore_barrier()

    pltpu.sync_copy(val_v, w_sh.at[idx_v], add=True)
    plsc.subcore_barrier()

    pltpu.sync_copy(w_sh.at[pl.ds(s * SLICE, SLICE)],
                    out_hbm.at[c, pl.ds(s * SLICE, SLICE)])


@functools.partial(
    pl.kernel,
    out_type=jax.ShapeDtypeStruct((NC, L * L), jnp.float32),
    mesh=plsc.VectorSubcoreMesh(core_axis_name="c", subcore_axis_name="s"),
    scratch_types=[
        pltpu.VMEM((EPC,), jnp.int32),
        pltpu.VMEM((EPC,), jnp.int32),
        pltpu.VMEM((EPC,), jnp.float32),
        pltpu.VMEM((EPC,), jnp.int32),
        pltpu.VMEM((EPC,), jnp.int32),
        pltpu.VMEM((EPC,), jnp.float32),
        pltpu.VMEM((EPC,), jnp.float32),
        pltpu.VMEM((EPC,), jnp.int32),
        pltpu.VMEM((ZB,), jnp.float32),
        pltpu.VMEM_SHARED((L * L,), jnp.float32),
    ],
)
def _build_w(d1_hbm, s1_hbm, w1_hbm, d2_hbm, s2_hbm, w2_hbm, out_hbm,
             d1_v, s1_v, v1_v, d2_v, s2_v, v2_v, val_v, idx_v, zbuf_v, w_sh):
    _build_w_body(d1_hbm, s1_hbm, w1_hbm, d2_hbm, s2_hbm, w2_hbm, out_hbm,
                  d1_v, s1_v, v1_v, d2_v, s2_v, v2_v, val_v, idx_v, zbuf_v, w_sh)


def _mlp_body(x_ref, w_ref, b1_ref, b2_ref, o_ref):
    h = jnp.dot(x_ref[...], w_ref[0], preferred_element_type=jnp.float32)
    h = jnp.maximum(h + b1_ref[...], 0.0)
    o_ref[...] = jnp.dot(h, w_ref[1], preferred_element_type=jnp.float32) + b2_ref[...]


def kernel(input_tensor, edge_index1, weights1, bias1,
           edge_index2, weights2, bias2):
    ei1 = edge_index1.astype(jnp.int32)
    ei2 = edge_index2.astype(jnp.int32)
    w12 = _build_w(ei1[0], ei1[1], weights1,
                   ei2[0], ei2[1], weights2).reshape(NC, L, L)

    out = pl.pallas_call(
        _mlp_body,
        out_shape=jax.ShapeDtypeStruct((input_tensor.shape[0], L), jnp.float32),
    )(input_tensor, w12, bias1.reshape(1, L), bias2.reshape(1, L))
    return out


# speedup vs baseline: 1.1422x; 1.0996x over previous
import functools

import jax
import jax.numpy as jnp
from jax import lax
from jax.experimental import pallas as pl
from jax.experimental.pallas import tpu as pltpu
from jax.experimental.pallas import tpu_sc as plsc

L = 1024
E = 65536
NC, NS = 2, 16
EPC = E // NS
CHUNK = 128
NCHUNK = EPC // CHUNK
SLICE = (L * L) // NS
ZB = 8192


def _build_w_body(d1_hbm, s1_hbm, w1_hbm, d2_hbm, s2_hbm, w2_hbm, out_hbm,
                  d1_v, s1_v, v1_v, d2_v, s2_v, v2_v, val_v, idx_v, zbuf_v,
                  sem_a, sem_b, w_sh):
    c = lax.axis_index("c")
    s = lax.axis_index("s")
    base = s * EPC

    stage = [
        pltpu.async_copy(d1_hbm.at[pl.ds(base, EPC)], d1_v, sem_a),
        pltpu.async_copy(s1_hbm.at[pl.ds(base, EPC)], s1_v, sem_a),
        pltpu.async_copy(w1_hbm.at[pl.ds(base, EPC)], v1_v, sem_a),
        pltpu.async_copy(d2_hbm.at[pl.ds(base, EPC)], d2_v, sem_a),
        pltpu.async_copy(s2_hbm.at[pl.ds(base, EPC)], s2_v, sem_a),
        pltpu.async_copy(w2_hbm.at[pl.ds(base, EPC)], v2_v, sem_a),
    ]

    for k in range(ZB // 16):
        zbuf_v[pl.ds(k * 16, 16)] = jnp.zeros((16,), jnp.float32)
    zero = [pltpu.async_copy(zbuf_v, w_sh.at[pl.ds(s * SLICE + k * ZB, ZB)],
                             sem_b)
            for k in range(SLICE // ZB)]
    for cp in stage:
        cp.wait()

    on1 = c == 0
    for t in range(0, EPC, 16):
        srcv = jnp.where(on1, s1_v[pl.ds(t, 16)], s2_v[pl.ds(t, 16)])
        dstv = jnp.where(on1, d1_v[pl.ds(t, 16)], d2_v[pl.ds(t, 16)])
        idx_v[pl.ds(t, 16)] = srcv * L + dstv
        val_v[pl.ds(t, 16)] = jnp.where(
            on1, v1_v[pl.ds(t, 16)], v2_v[pl.ds(t, 16)])
    for cp in zero:
        cp.wait()
    plsc.subcore_barrier()

    pltpu.sync_copy(val_v, w_sh.at[idx_v], add=True)
    plsc.subcore_barrier()

    pltpu.sync_copy(w_sh.at[pl.ds(s * SLICE, SLICE)],
                    out_hbm.at[c, pl.ds(s * SLICE, SLICE)])


@functools.partial(
    pl.kernel,
    out_type=jax.ShapeDtypeStruct((NC, L * L), jnp.float32),
    mesh=plsc.VectorSubcoreMesh(core_axis_name="c", subcore_axis_name="s"),
    scratch_types=[
        pltpu.VMEM((EPC,), jnp.int32),
        pltpu.VMEM((EPC,), jnp.int32),
        pltpu.VMEM((EPC,), jnp.float32),
        pltpu.VMEM((EPC,), jnp.int32),
        pltpu.VMEM((EPC,), jnp.int32),
        pltpu.VMEM((EPC,), jnp.float32),
        pltpu.VMEM((EPC,), jnp.float32),
        pltpu.VMEM((EPC,), jnp.int32),
        pltpu.VMEM((ZB,), jnp.float32),
        pltpu.SemaphoreType.DMA,
        pltpu.SemaphoreType.DMA,
        pltpu.VMEM_SHARED((L * L,), jnp.float32),
    ],
)
def _build_w(d1_hbm, s1_hbm, w1_hbm, d2_hbm, s2_hbm, w2_hbm, out_hbm,
             d1_v, s1_v, v1_v, d2_v, s2_v, v2_v, val_v, idx_v, zbuf_v,
             sem_a, sem_b, w_sh):
    _build_w_body(d1_hbm, s1_hbm, w1_hbm, d2_hbm, s2_hbm, w2_hbm, out_hbm,
                  d1_v, s1_v, v1_v, d2_v, s2_v, v2_v, val_v, idx_v, zbuf_v,
                  sem_a, sem_b, w_sh)


def _mlp_body(x_ref, w_ref, b1_ref, b2_ref, o_ref):
    h = jnp.dot(x_ref[...], w_ref[0], preferred_element_type=jnp.float32)
    h = jnp.maximum(h + b1_ref[...], 0.0)
    o_ref[...] = jnp.dot(h, w_ref[1], preferred_element_type=jnp.float32) + b2_ref[...]


def kernel(input_tensor, edge_index1, weights1, bias1,
           edge_index2, weights2, bias2):
    ei1 = edge_index1.astype(jnp.int32)
    ei2 = edge_index2.astype(jnp.int32)
    w12 = _build_w(ei1[0], ei1[1], weights1,
                   ei2[0], ei2[1], weights2).reshape(NC, L, L)

    out = pl.pallas_call(
        _mlp_body,
        out_shape=jax.ShapeDtypeStruct((input_tensor.shape[0], L), jnp.float32),
    )(input_tensor, w12, bias1.reshape(1, L), bias2.reshape(1, L))
    return out
